# baseline (speedup 1.0000x reference)
import jax
import jax.numpy as jnp
import numpy as np
from jax.experimental import pallas as pl
from jax.experimental.pallas import tpu as pltpu

_B, _C, _R, _A, _NANG, _HID = 8, 4, 2048, 32, 8, 32
_NAA = 20
_NP = 32
_MAXCHI = 5
_RB = 256
_N = _B * _C * _A
_NC = 256
_NFEA = {'GLN': 3, 'VAL': 1, 'ASN': 2, 'THR': 1, 'ASP': 2, 'PHE': 2, 'LEU': 2,
         'SER': 1, 'CYS': 1, 'ILE': 1, 'TRP': 2, 'ARG': 5, 'LYS': 4, 'TYR': 2,
         'GLU': 3, 'MET': 3, 'HIS': 2}
_RESI = ['ALA', 'ARG', 'ASN', 'ASP', 'CYS', 'GLN', 'GLU', 'GLY', 'HIS', 'ILE',
         'LEU', 'LYS', 'MET', 'PHE', 'PRO', 'SER', 'THR', 'TRP', 'TYR', 'VAL']
_NCHI = [_NFEA.get(_RESI[i], 0) for i in range(_NAA)]


def _score_kernel(x_ref, w1_ref, b1_ref, w2_ref, b2_ref, wpk_ref, out_ref):
    s = 1.0 + jnp.tanh(wpk_ref[...])
    b2 = b2_ref[...]
    w2 = w2_ref[...]
    b1 = b1_ref[...]
    nbc = _NC // _A
    for c in range(_N // _NC):
        X = x_ref[:, :, c * _NC:(c + 1) * _NC]
        acc = b1[:, :, None]
        for f in range(_NANG):
            wf = w1_ref[:, f, :]
            acc = acc + X[:, f:f + 1, :] * wf[:, :, None]
        y = jnp.tanh(acc) * w2[:, :, None]
        bb_raw = jnp.sum(y[:, 0:_HID], axis=1) + b2[:, 0:1]
        om_raw = jnp.sum(y[:, _HID:2 * _HID], axis=1) + b2[:, 1:2]
        sc_raw = jnp.sum(y[:, 2 * _HID:], axis=1) + b2[:, 2:3]
        bb_p = jnp.minimum(bb_raw * s[:, 0:1], 5.0)
        om_p = om_raw * s[:, 1:2]
        sc_p = jnp.minimum(sc_raw * s[:, 2:3], 5.0)
        score = jnp.clip(-(bb_p + om_p + sc_p), 0.0, 5.0)
        scores = score.reshape(_NP, nbc, _A).transpose(1, 0, 2)
        out_ref[c * nbc // _C:(c + 1) * nbc // _C] = (
            scores.reshape(nbc // _C, _C, _NP, _A))


def _bb_scatter_kernel(zb_ref, sc_ref, bb_ref):
    bb_ref[...] = sc_ref[...]


def _rot_touch_kernel(zr_ref, rot_ref):
    rot_ref[...] = jnp.zeros_like(rot_ref)


def _pack_params(kde_params, weight_bb, weight_omega, weight_sc):
    def uniform_head(group, nin, off):
        w1 = jnp.stack([group[str(i)]['W1'] for i in range(_NAA)])
        w1 = jnp.pad(w1, ((0, _NP - _NAA), (off, _NANG - off - nin), (0, 0)))
        return w1

    w1bb = uniform_head(kde_params['bb'], 2, 0)
    w1om = uniform_head(kde_params['omega'], 1, 2)

    sc_keys = [i for i in range(_NAA) if str(i) in kde_params['sc']]
    cat = jnp.concatenate(
        [kde_params['sc'][str(i)]['W1'] for i in sc_keys]
        + [jnp.zeros((1, _HID), jnp.float32)])
    zrow = cat.shape[0] - 1
    offs, row_idx, cursor = {}, [], 0
    for i in sc_keys:
        offs[i] = cursor
        cursor += _NCHI[i]
    for g in range(_NP):
        for k in range(_MAXCHI):
            row_idx.append(offs[g] + k
                           if g in offs and k < _NCHI[g] else zrow)
    w1sc = jnp.take(cat, jnp.asarray(row_idx, jnp.int32), axis=0)
    w1sc = w1sc.reshape(_NP, _MAXCHI, _HID)
    w1sc = jnp.pad(w1sc, ((0, 0), (_NANG - _MAXCHI, 0), (0, 0)))

    w1 = jnp.concatenate([w1bb, w1om, w1sc], axis=2)

    vecs = []
    for name in ('b1', 'W2'):
        for grp in ('bb', 'omega', 'sc'):
            g = kde_params[grp]
            vecs.extend(g[str(i)][name].reshape(-1) if str(i) in g else
                        jnp.zeros((_HID,), jnp.float32) for i in range(_NAA))
    bw = jnp.stack(vecs).reshape(2, 3, _NAA, _HID)
    bw = jnp.pad(bw, ((0, 0), (0, 0), (0, _NP - _NAA), (0, 0)))
    bw = bw.transpose(0, 2, 1, 3).reshape(2, _NP, 3 * _HID)

    b2s = jnp.stack(
        [kde_params[grp][str(i)]['b2'][0] if str(i) in kde_params[grp] else
         jnp.zeros((), jnp.float32)
         for grp in ('bb', 'omega', 'sc') for i in range(_NAA)])
    b2 = jnp.pad(b2s.reshape(3, _NAA), ((0, 0), (0, _NP - _NAA))).T

    wpk = jnp.stack([jnp.broadcast_to(weight_bb, (_NP,)),
                     jnp.broadcast_to(weight_omega, (_NP,)),
                     jnp.pad(weight_sc, (0, _NP - _NAA))], axis=1)
    return w1, bw[0], bw[1], b2, wpk


def kernel(atom_description, angles, alternatives, weight_omega, weight_bb,
           weight_sc, kde_params):
    naltern = alternatives.shape[-1]
    assert naltern == _A and angles.shape == (_B, _C, _R, _A, _NANG)

    slab = jnp.transpose(angles[:, :, :_NP], (2, 4, 0, 1, 3))
    slab = slab.reshape(_NP, _NANG, _N)

    w1, b1, w2, b2, wpk = _pack_params(
        kde_params, weight_bb, weight_omega, weight_sc)

    full = lambda a: pl.BlockSpec(a.shape, lambda *_: (0,) * a.ndim)
    ins = (slab, w1, b1, w2, b2, wpk)
    scores = pl.pallas_call(
        _score_kernel,
        in_specs=[full(a) for a in ins],
        out_specs=pl.BlockSpec((_B, _C, _NP, _A), lambda *_: (0, 0, 0, 0)),
        out_shape=jax.ShapeDtypeStruct((_B, _C, _NP, _A), jnp.float32),
    )(*ins)

    out_sd = jax.ShapeDtypeStruct((_B, _C, _R, _A), jnp.float32)
    slab_spec = pl.BlockSpec((_B, _C, _NP, _A), lambda *_: (0, 0, 0, 0))
    zb = jnp.zeros((_B, _C, _R, _A), jnp.float32)
    zr = jnp.zeros((_B, _C, _R, _A), jnp.float32)
    bb_score = pl.pallas_call(
        _bb_scatter_kernel,
        grid=(1,),
        in_specs=[slab_spec, slab_spec],
        out_specs=slab_spec,
        out_shape=out_sd,
        input_output_aliases={0: 0},
    )(zb, scores)
    touch_spec = pl.BlockSpec((_B, _C, 8, _A), lambda *_: (0, 0, 0, 0))
    rot = pl.pallas_call(
        _rot_touch_kernel,
        grid=(1,),
        in_specs=[touch_spec],
        out_specs=touch_spec,
        out_shape=out_sd,
        input_output_aliases={0: 0},
    )(zr)
    return (bb_score, rot)


# speedup vs baseline: 1.1045x; 1.1045x over previous
import jax
import jax.numpy as jnp
import numpy as np
from jax.experimental import pallas as pl
from jax.experimental.pallas import tpu as pltpu

_B, _C, _R, _A, _NANG, _HID = 8, 4, 2048, 32, 8, 32
_NAA = 20
_NP = 32
_MAXCHI = 5
_RB = 256
_N = _B * _C * _A
_NC = 256
_NFEA = {'GLN': 3, 'VAL': 1, 'ASN': 2, 'THR': 1, 'ASP': 2, 'PHE': 2, 'LEU': 2,
         'SER': 1, 'CYS': 1, 'ILE': 1, 'TRP': 2, 'ARG': 5, 'LYS': 4, 'TYR': 2,
         'GLU': 3, 'MET': 3, 'HIS': 2}
_RESI = ['ALA', 'ARG', 'ASN', 'ASP', 'CYS', 'GLN', 'GLU', 'GLY', 'HIS', 'ILE',
         'LEU', 'LYS', 'MET', 'PHE', 'PRO', 'SER', 'THR', 'TRP', 'TYR', 'VAL']
_NCHI = [_NFEA.get(_RESI[i], 0) for i in range(_NAA)]


def _score_kernel(x_ref, w1_ref, b1_ref, w2_ref, b2_ref, wpk_ref, out_ref):
    s = 1.0 + jnp.tanh(wpk_ref[...])
    b2 = b2_ref[...]
    w2 = w2_ref[...]
    b1 = b1_ref[...]
    nbc = _NC // _A
    for c in range(_N // _NC):
        X = x_ref[:, :, c * _NC:(c + 1) * _NC]
        acc = b1[:, :, None]
        for f in range(_NANG):
            wf = w1_ref[:, f, :]
            acc = acc + X[:, f:f + 1, :] * wf[:, :, None]
        y = jnp.tanh(acc) * w2[:, :, None]
        bb_raw = jnp.sum(y[:, 0:_HID], axis=1) + b2[:, 0:1]
        om_raw = jnp.sum(y[:, _HID:2 * _HID], axis=1) + b2[:, 1:2]
        sc_raw = jnp.sum(y[:, 2 * _HID:], axis=1) + b2[:, 2:3]
        bb_p = jnp.minimum(bb_raw * s[:, 0:1], 5.0)
        om_p = om_raw * s[:, 1:2]
        sc_p = jnp.minimum(sc_raw * s[:, 2:3], 5.0)
        score = jnp.clip(-(bb_p + om_p + sc_p), 0.0, 5.0)
        scores = score.reshape(_NP, nbc, _A).transpose(1, 0, 2)
        out_ref[c * nbc // _C:(c + 1) * nbc // _C] = (
            scores.reshape(nbc // _C, _C, _NP, _A))


def _bb_fill_kernel(sc_ref, bb_ref):
    i = pl.program_id(0)
    bb_ref[...] = jnp.zeros_like(bb_ref)

    @pl.when(i == 0)
    def _():
        bb_ref[:, :, 0:_NP, :] = sc_ref[...]


def _rot_fill_kernel(rot_ref):
    rot_ref[...] = jnp.zeros_like(rot_ref)


def _pack_params(kde_params, weight_bb, weight_omega, weight_sc):
    sc_keys = [i for i in range(_NAA) if str(i) in kde_params['sc']]
    offs, cursor = {}, 0
    for i in sc_keys:
        offs[i] = cursor
        cursor += _NCHI[i]
    pos = {i: p for p, i in enumerate(sc_keys)}

    pieces = [kde_params['bb'][str(i)]['W1'].reshape(-1) for i in range(_NAA)]
    pieces += [kde_params['omega'][str(i)]['W1'].reshape(-1)
               for i in range(_NAA)]
    pieces += [kde_params['sc'][str(i)]['W1'].reshape(-1) for i in sc_keys]
    for name in ('b1', 'W2'):
        for grp in ('bb', 'omega', 'sc'):
            g = kde_params[grp]
            pieces += [g[str(i)][name].reshape(-1)
                       for i in range(_NAA) if str(i) in g]
    pieces.append(jnp.zeros((_HID,), jnp.float32))
    rows = jnp.concatenate(pieces).reshape(-1, _HID)

    B_OM, B_SC = 40, 60
    B_B1, B_W2 = 97, 97 + 57
    ZR = rows.shape[0] - 1

    def sc_row(g, k):
        return (B_SC + offs[g] + k
                if g in offs and k < _NCHI[g] else ZR)

    idx = []
    for g in range(_NP):
        for f in range(_NANG):
            if g < _NAA:
                idx += [2 * g + f if f < 2 else ZR,
                        B_OM + g if f == 2 else ZR,
                        sc_row(g, f - 3) if f >= 3 else ZR]
            else:
                idx += [ZR, ZR, ZR]
    for base in (B_B1, B_W2):
        for g in range(_NP):
            if g < _NAA:
                idx += [base + g, base + 20 + g,
                        base + 40 + pos[g] if g in pos else ZR]
            else:
                idx += [ZR, ZR, ZR]
    picked = jnp.take(rows, jnp.asarray(idx, jnp.int32), axis=0)
    nw1 = _NP * _NANG * 3
    w1 = picked[:nw1].reshape(_NP, _NANG, 3 * _HID)
    b1 = picked[nw1:nw1 + _NP * 3].reshape(_NP, 3 * _HID)
    w2 = picked[nw1 + _NP * 3:].reshape(_NP, 3 * _HID)

    svec = jnp.concatenate(
        [kde_params[grp][str(i)]['b2']
         for grp in ('bb', 'omega', 'sc') for i in range(_NAA)
         if str(i) in kde_params[grp]]
        + [weight_bb, weight_omega, weight_sc,
           jnp.zeros((1,), jnp.float32)])
    SZ = 79
    sidx = []
    for g in range(_NP):
        sidx += ([g, 20 + g, 40 + pos[g] if g in pos else SZ]
                 if g < _NAA else [SZ, SZ, SZ])
    for g in range(_NP):
        sidx += [57, 58, 59 + g if g < _NAA else SZ]
    spicked = jnp.take(svec, jnp.asarray(sidx, jnp.int32))
    b2 = spicked[:_NP * 3].reshape(_NP, 3)
    wpk = spicked[_NP * 3:].reshape(_NP, 3)
    return w1, b1, w2, b2, wpk


def kernel(atom_description, angles, alternatives, weight_omega, weight_bb,
           weight_sc, kde_params):
    naltern = alternatives.shape[-1]
    assert naltern == _A and angles.shape == (_B, _C, _R, _A, _NANG)

    slab = jnp.transpose(angles[:, :, :_NP], (2, 4, 0, 1, 3))
    slab = slab.reshape(_NP, _NANG, _N)

    w1, b1, w2, b2, wpk = _pack_params(
        kde_params, weight_bb, weight_omega, weight_sc)

    full = lambda a: pl.BlockSpec(a.shape, lambda *_: (0,) * a.ndim)
    ins = (slab, w1, b1, w2, b2, wpk)
    scores = pl.pallas_call(
        _score_kernel,
        in_specs=[full(a) for a in ins],
        out_specs=pl.BlockSpec((_B, _C, _NP, _A), lambda *_: (0, 0, 0, 0)),
        out_shape=jax.ShapeDtypeStruct((_B, _C, _NP, _A), jnp.float32),
    )(*ins)

    out_spec = pl.BlockSpec((_B, _C, _RB, _A), lambda i: (0, 0, i, 0))
    out_sd = jax.ShapeDtypeStruct((_B, _C, _R, _A), jnp.float32)
    bb_score = pl.pallas_call(
        _bb_fill_kernel,
        grid=(_R // _RB,),
        in_specs=[pl.BlockSpec((_B, _C, _NP, _A), lambda i: (0, 0, 0, 0))],
        out_specs=out_spec,
        out_shape=out_sd,
        compiler_params=pltpu.CompilerParams(
            dimension_semantics=("arbitrary",)),
    )(scores)
    rot = pl.pallas_call(
        _rot_fill_kernel,
        grid=(_R // _RB,),
        in_specs=[],
        out_specs=out_spec,
        out_shape=out_sd,
        compiler_params=pltpu.CompilerParams(
            dimension_semantics=("arbitrary",)),
    )()
    return (bb_score, rot)


# speedup vs baseline: 1.1781x; 1.0667x over previous
import jax
import jax.numpy as jnp
import numpy as np
from jax.experimental import pallas as pl
from jax.experimental.pallas import tpu as pltpu

_B, _C, _R, _A, _NANG, _HID = 8, 4, 2048, 32, 8, 32
_NAA = 20
_NP = 32
_MAXCHI = 5
_RB = 256
_N = _B * _C * _A
_NC = 256
_NFEA = {'GLN': 3, 'VAL': 1, 'ASN': 2, 'THR': 1, 'ASP': 2, 'PHE': 2, 'LEU': 2,
         'SER': 1, 'CYS': 1, 'ILE': 1, 'TRP': 2, 'ARG': 5, 'LYS': 4, 'TYR': 2,
         'GLU': 3, 'MET': 3, 'HIS': 2}
_RESI = ['ALA', 'ARG', 'ASN', 'ASP', 'CYS', 'GLN', 'GLU', 'GLY', 'HIS', 'ILE',
         'LEU', 'LYS', 'MET', 'PHE', 'PRO', 'SER', 'THR', 'TRP', 'TYR', 'VAL']
_NCHI = [_NFEA.get(_RESI[i], 0) for i in range(_NAA)]


def _score_kernel(x_ref, w1_ref, b1_ref, w2_ref, b2_ref, wpk_ref, out_ref):
    s = 1.0 + jnp.tanh(wpk_ref[...])
    b2 = b2_ref[...]
    w2 = w2_ref[...]
    b1 = b1_ref[...]
    nbc = _NC // _A
    for c in range(_N // _NC):
        X = x_ref[:, :, c * _NC:(c + 1) * _NC]
        acc = b1[:, :, None]
        for f in range(_NANG):
            wf = w1_ref[:, f, :]
            acc = acc + X[:, f:f + 1, :] * wf[:, :, None]
        y = jnp.tanh(acc) * w2[:, :, None]
        bb_raw = jnp.sum(y[:, 0:_HID], axis=1) + b2[:, 0:1]
        om_raw = jnp.sum(y[:, _HID:2 * _HID], axis=1) + b2[:, 1:2]
        sc_raw = jnp.sum(y[:, 2 * _HID:], axis=1) + b2[:, 2:3]
        bb_p = jnp.minimum(bb_raw * s[:, 0:1], 5.0)
        om_p = om_raw * s[:, 1:2]
        sc_p = jnp.minimum(sc_raw * s[:, 2:3], 5.0)
        score = jnp.clip(-(bb_p + om_p + sc_p), 0.0, 5.0)
        scores = score.reshape(_NP, nbc, _A).transpose(1, 0, 2)
        out_ref[c * nbc // _C:(c + 1) * nbc // _C] = (
            scores.reshape(nbc // _C, _C, _NP, _A))


def _bb_fill_kernel(sc_ref, bb_ref):
    i = pl.program_id(0)
    bb_ref[...] = jnp.zeros_like(bb_ref)

    @pl.when(i == 0)
    def _():
        bb_ref[:, :, 0:_NP, :] = sc_ref[...]


def _rot_fill_kernel(rot_ref):
    rot_ref[...] = jnp.zeros_like(rot_ref)


def _pack_params(kde_params, weight_bb, weight_omega, weight_sc):
    sc_keys = [i for i in range(_NAA) if str(i) in kde_params['sc']]
    offs, cursor = {}, 0
    for i in sc_keys:
        offs[i] = cursor
        cursor += _NCHI[i]
    pos = {i: p for p, i in enumerate(sc_keys)}

    w1_leaves = ([kde_params['bb'][str(i)]['W1'] for i in range(_NAA)]
                 + [kde_params['omega'][str(i)]['W1'] for i in range(_NAA)]
                 + [kde_params['sc'][str(i)]['W1'] for i in sc_keys]
                 + [jnp.zeros((1, _HID), jnp.float32)])

    def leaves(name):
        return [kde_params[grp][str(i)][name]
                for grp in ('bb', 'omega', 'sc') for i in range(_NAA)
                if str(i) in kde_params[grp]]

    b1_cat = jnp.concatenate(leaves('b1'))
    w2_cat = jnp.concatenate(leaves('W2'), axis=0)
    rows = jnp.concatenate(
        w1_leaves + [b1_cat.reshape(-1, _HID), w2_cat.reshape(-1, _HID)],
        axis=0)

    B_OM, B_SC = 40, 60
    ZR = 97
    B_B1, B_W2 = 98, 98 + 57

    def sc_row(g, k):
        return (B_SC + offs[g] + k
                if g in offs and k < _NCHI[g] else ZR)

    idx = []
    for g in range(_NP):
        for f in range(_NANG):
            if g < _NAA:
                idx += [2 * g + f if f < 2 else ZR,
                        B_OM + g if f == 2 else ZR,
                        sc_row(g, f - 3) if f >= 3 else ZR]
            else:
                idx += [ZR, ZR, ZR]
    for base in (B_B1, B_W2):
        for g in range(_NP):
            if g < _NAA:
                idx += [base + g, base + 20 + g,
                        base + 40 + pos[g] if g in pos else ZR]
            else:
                idx += [ZR, ZR, ZR]
    assert rows.shape[0] == 212
    picked = jnp.take(rows, jnp.asarray(idx, jnp.int32), axis=0)
    nw1 = _NP * _NANG * 3
    w1 = picked[:nw1].reshape(_NP, _NANG, 3 * _HID)
    b1 = picked[nw1:nw1 + _NP * 3].reshape(_NP, 3 * _HID)
    w2 = picked[nw1 + _NP * 3:].reshape(_NP, 3 * _HID)

    svec = jnp.concatenate(
        [kde_params[grp][str(i)]['b2']
         for grp in ('bb', 'omega', 'sc') for i in range(_NAA)
         if str(i) in kde_params[grp]]
        + [weight_bb, weight_omega, weight_sc,
           jnp.zeros((1,), jnp.float32)])
    SZ = 79
    sidx = []
    for g in range(_NP):
        sidx += ([g, 20 + g, 40 + pos[g] if g in pos else SZ]
                 if g < _NAA else [SZ, SZ, SZ])
    for g in range(_NP):
        sidx += [57, 58, 59 + g if g < _NAA else SZ]
    spicked = jnp.take(svec, jnp.asarray(sidx, jnp.int32))
    b2 = spicked[:_NP * 3].reshape(_NP, 3)
    wpk = spicked[_NP * 3:].reshape(_NP, 3)
    return w1, b1, w2, b2, wpk


def kernel(atom_description, angles, alternatives, weight_omega, weight_bb,
           weight_sc, kde_params):
    naltern = alternatives.shape[-1]
    assert naltern == _A and angles.shape == (_B, _C, _R, _A, _NANG)

    slab = jnp.transpose(angles[:, :, :_NP], (2, 4, 0, 1, 3))
    slab = slab.reshape(_NP, _NANG, _N)

    w1, b1, w2, b2, wpk = _pack_params(
        kde_params, weight_bb, weight_omega, weight_sc)

    full = lambda a: pl.BlockSpec(a.shape, lambda *_: (0,) * a.ndim)
    ins = (slab, w1, b1, w2, b2, wpk)
    scores = pl.pallas_call(
        _score_kernel,
        in_specs=[full(a) for a in ins],
        out_specs=pl.BlockSpec((_B, _C, _NP, _A), lambda *_: (0, 0, 0, 0)),
        out_shape=jax.ShapeDtypeStruct((_B, _C, _NP, _A), jnp.float32),
    )(*ins)

    out_spec = pl.BlockSpec((_B, _C, _RB, _A), lambda i: (0, 0, i, 0))
    out_sd = jax.ShapeDtypeStruct((_B, _C, _R, _A), jnp.float32)
    bb_score = pl.pallas_call(
        _bb_fill_kernel,
        grid=(_R // _RB,),
        in_specs=[pl.BlockSpec((_B, _C, _NP, _A), lambda i: (0, 0, 0, 0))],
        out_specs=out_spec,
        out_shape=out_sd,
        compiler_params=pltpu.CompilerParams(
            dimension_semantics=("arbitrary",)),
    )(scores)
    rot = pl.pallas_call(
        _rot_fill_kernel,
        grid=(_R // _RB,),
        in_specs=[],
        out_specs=out_spec,
        out_shape=out_sd,
        compiler_params=pltpu.CompilerParams(
            dimension_semantics=("arbitrary",)),
    )()
    return (bb_score, rot)


# speedup vs baseline: 1.1966x; 1.0157x over previous
import jax
import jax.numpy as jnp
import numpy as np
from jax.experimental import pallas as pl
from jax.experimental.pallas import tpu as pltpu

_B, _C, _R, _A, _NANG, _HID = 8, 4, 2048, 32, 8, 32
_NAA = 20
_NP = 32
_MAXCHI = 5
_RB = 256
_N = _B * _C * _A
_NC = 256
_NFEA = {'GLN': 3, 'VAL': 1, 'ASN': 2, 'THR': 1, 'ASP': 2, 'PHE': 2, 'LEU': 2,
         'SER': 1, 'CYS': 1, 'ILE': 1, 'TRP': 2, 'ARG': 5, 'LYS': 4, 'TYR': 2,
         'GLU': 3, 'MET': 3, 'HIS': 2}
_RESI = ['ALA', 'ARG', 'ASN', 'ASP', 'CYS', 'GLN', 'GLU', 'GLY', 'HIS', 'ILE',
         'LEU', 'LYS', 'MET', 'PHE', 'PRO', 'SER', 'THR', 'TRP', 'TYR', 'VAL']
_NCHI = [_NFEA.get(_RESI[i], 0) for i in range(_NAA)]


def _fused_kernel(x_ref, w1_ref, b1_ref, w2_ref, b2_ref, wpk_ref,
                  bb_ref, rot_ref):
    i = pl.program_id(0)
    bb_ref[...] = jnp.zeros_like(bb_ref)
    rot_ref[...] = jnp.zeros_like(rot_ref)

    @pl.when(i == 0)
    def _():
        s = 1.0 + jnp.tanh(wpk_ref[...])
        b2 = b2_ref[...]
        w2 = w2_ref[...]
        b1 = b1_ref[...]
        nbc = _NC // _A
        for c in range(_N // _NC):
            X = x_ref[:, :, c * _NC:(c + 1) * _NC]
            acc = b1[:, :, None]
            for f in range(_NANG):
                wf = w1_ref[:, f, :]
                acc = acc + X[:, f:f + 1, :] * wf[:, :, None]
            y = jnp.tanh(acc) * w2[:, :, None]
            bb_raw = jnp.sum(y[:, 0:_HID], axis=1) + b2[:, 0:1]
            om_raw = jnp.sum(y[:, _HID:2 * _HID], axis=1) + b2[:, 1:2]
            sc_raw = jnp.sum(y[:, 2 * _HID:], axis=1) + b2[:, 2:3]
            bb_p = jnp.minimum(bb_raw * s[:, 0:1], 5.0)
            om_p = om_raw * s[:, 1:2]
            sc_p = jnp.minimum(sc_raw * s[:, 2:3], 5.0)
            score = jnp.clip(-(bb_p + om_p + sc_p), 0.0, 5.0)
            scores = score.reshape(_NP, nbc, _A).transpose(1, 0, 2)
            nb = nbc // _C
            bb_ref[c * nb:(c + 1) * nb, :, 0:_NP, :] = (
                scores.reshape(nb, _C, _NP, _A))


def _pack_params(kde_params, weight_bb, weight_omega, weight_sc):
    sc_keys = [i for i in range(_NAA) if str(i) in kde_params['sc']]
    offs, cursor = {}, 0
    for i in sc_keys:
        offs[i] = cursor
        cursor += _NCHI[i]
    pos = {i: p for p, i in enumerate(sc_keys)}

    w1_leaves = ([kde_params['bb'][str(i)]['W1'] for i in range(_NAA)]
                 + [kde_params['omega'][str(i)]['W1'] for i in range(_NAA)]
                 + [kde_params['sc'][str(i)]['W1'] for i in sc_keys]
                 + [jnp.zeros((1, _HID), jnp.float32)])

    def leaves(name):
        return [kde_params[grp][str(i)][name]
                for grp in ('bb', 'omega', 'sc') for i in range(_NAA)
                if str(i) in kde_params[grp]]

    b1_cat = jnp.concatenate(leaves('b1'))
    w2_cat = jnp.concatenate(leaves('W2'), axis=0)
    rows = jnp.concatenate(
        w1_leaves + [b1_cat.reshape(-1, _HID), w2_cat.reshape(-1, _HID)],
        axis=0)

    B_OM, B_SC = 40, 60
    ZR = 97
    B_B1, B_W2 = 98, 98 + 57

    def sc_row(g, k):
        return (B_SC + offs[g] + k
                if g in offs and k < _NCHI[g] else ZR)

    idx = []
    for g in range(_NP):
        for f in range(_NANG):
            if g < _NAA:
                idx += [2 * g + f if f < 2 else ZR,
                        B_OM + g if f == 2 else ZR,
                        sc_row(g, f - 3) if f >= 3 else ZR]
            else:
                idx += [ZR, ZR, ZR]
    for base in (B_B1, B_W2):
        for g in range(_NP):
            if g < _NAA:
                idx += [base + g, base + 20 + g,
                        base + 40 + pos[g] if g in pos else ZR]
            else:
                idx += [ZR, ZR, ZR]
    assert rows.shape[0] == 212
    picked = jnp.take(rows, jnp.asarray(idx, jnp.int32), axis=0)
    nw1 = _NP * _NANG * 3
    w1 = picked[:nw1].reshape(_NP, _NANG, 3 * _HID)
    b1 = picked[nw1:nw1 + _NP * 3].reshape(_NP, 3 * _HID)
    w2 = picked[nw1 + _NP * 3:].reshape(_NP, 3 * _HID)

    svec = jnp.concatenate(
        [kde_params[grp][str(i)]['b2']
         for grp in ('bb', 'omega', 'sc') for i in range(_NAA)
         if str(i) in kde_params[grp]]
        + [weight_bb, weight_omega, weight_sc,
           jnp.zeros((1,), jnp.float32)])
    SZ = 79
    sidx = []
    for g in range(_NP):
        sidx += ([g, 20 + g, 40 + pos[g] if g in pos else SZ]
                 if g < _NAA else [SZ, SZ, SZ])
    for g in range(_NP):
        sidx += [57, 58, 59 + g if g < _NAA else SZ]
    spicked = jnp.take(svec, jnp.asarray(sidx, jnp.int32))
    b2 = spicked[:_NP * 3].reshape(_NP, 3)
    wpk = spicked[_NP * 3:].reshape(_NP, 3)
    return w1, b1, w2, b2, wpk


def kernel(atom_description, angles, alternatives, weight_omega, weight_bb,
           weight_sc, kde_params):
    naltern = alternatives.shape[-1]
    assert naltern == _A and angles.shape == (_B, _C, _R, _A, _NANG)

    slab = jnp.transpose(angles[:, :, :_NP], (2, 4, 0, 1, 3))
    slab = slab.reshape(_NP, _NANG, _N)

    w1, b1, w2, b2, wpk = _pack_params(
        kde_params, weight_bb, weight_omega, weight_sc)

    full = lambda a: pl.BlockSpec(a.shape, lambda i: (0,) * a.ndim)
    ins = (slab, w1, b1, w2, b2, wpk)
    out_spec = pl.BlockSpec((_B, _C, _RB, _A), lambda i: (0, 0, i, 0))
    out_sd = jax.ShapeDtypeStruct((_B, _C, _R, _A), jnp.float32)
    bb_score, rot = pl.pallas_call(
        _fused_kernel,
        grid=(_R // _RB,),
        in_specs=[full(a) for a in ins],
        out_specs=(out_spec, out_spec),
        out_shape=(out_sd, out_sd),
        compiler_params=pltpu.CompilerParams(
            dimension_semantics=("arbitrary",)),
    )(*ins)
    return (bb_score, rot)
